# baseline (speedup 1.0000x reference)
import functools

import jax
import jax.numpy as jnp
from jax import lax
from jax.experimental import pallas as pl
from jax.experimental.pallas import tpu as pltpu
from jax.experimental.pallas import tpu_sc as plsc

HID = 768
EPS = 1e-5
NC = 2
NS = 16
NW = NC * NS
CH = 16
NSL = HID // 16
LANES = 16


def _rsqrt(x):
    i = plsc.bitcast(x, jnp.int32)
    i = 0x5F3759DF - lax.shift_right_logical(i, 1)
    y = plsc.bitcast(i, jnp.float32)
    for _ in range(3):
        y = y * (1.5 - 0.5 * x * y * y)
    return y


def _sc_body(seq, wtab, ids, ptab, gamma, beta, out,
             idx_v, rows_v, prow_v, gamma_v, beta_v, sem):
    wid = lax.axis_index("s") * NC + lax.axis_index("c")
    ntok = ids.shape[0]
    per_w = ntok // NW
    nchunk = per_w // CH
    base = wid * per_w

    pltpu.sync_copy(gamma, gamma_v)
    pltpu.sync_copy(beta, beta_v)

    def chunk_body(c, _):
        tok0 = base + c * CH
        pos0 = lax.rem(tok0, seq)
        pltpu.sync_copy(ids.at[pl.ds(tok0, CH)], idx_v)
        pltpu.async_copy(wtab.at[idx_v], rows_v, sem).wait()
        pltpu.sync_copy(ptab.at[pl.ds(pos0, CH)], prow_v)

        inv_h = jnp.float32(1.0 / HID)

        def tok_body(t, _):
            zero = jnp.zeros((LANES,), jnp.float32)

            def slice_acc(j, carry):
                a, a2 = carry
                sl = pl.ds(j * LANES, LANES)
                v = rows_v[t, sl] + prow_v[t, sl]
                rows_v[t, sl] = v
                return a + v, a2 + v * v

            a, a2 = lax.fori_loop(0, NSL, slice_acc, (zero, zero))
            mean = jnp.sum(a) * inv_h
            var = jnp.sum(a2) * inv_h - mean * mean
            r = jnp.max(_rsqrt(jnp.full((LANES,), var + EPS, jnp.float32)))

            def slice_norm(j, _):
                sl = pl.ds(j * LANES, LANES)
                e = rows_v[t, sl]
                rows_v[t, sl] = (e - mean) * r * gamma_v[sl] + beta_v[sl]
                return 0

            lax.fori_loop(0, NSL, slice_norm, 0)
            return 0

        lax.fori_loop(0, CH, tok_body, 0)

        pltpu.sync_copy(rows_v, out.at[pl.ds(tok0, CH)])
        return 0

    lax.fori_loop(0, nchunk, chunk_body, 0)


def kernel(input_ids, word_emb, pos_emb, type_emb, gamma, beta):
    b, seq = input_ids.shape
    ids = input_ids.reshape(b * seq).astype(jnp.int32)
    ptab = pos_emb[:seq] + type_emb[0]

    mesh = plsc.VectorSubcoreMesh(core_axis_name="c", subcore_axis_name="s",
                                  num_cores=NC, num_subcores=NS)
    k = pl.kernel(
        functools.partial(_sc_body, seq),
        out_type=jax.ShapeDtypeStruct((b * seq, HID), jnp.float32),
        mesh=mesh,
        compiler_params=pltpu.CompilerParams(needs_layout_passes=False),
        scratch_types=[
            pltpu.VMEM((CH,), jnp.int32),
            pltpu.VMEM((CH, HID), jnp.float32),
            pltpu.VMEM((CH, HID), jnp.float32),
            pltpu.VMEM((HID,), jnp.float32),
            pltpu.VMEM((HID,), jnp.float32),
            pltpu.SemaphoreType.DMA,
        ],
    )
    out = k(word_emb, ids, ptab, gamma, beta)
    return out.reshape(b, seq, HID)
